# baseline (speedup 1.0000x reference)
import functools

import jax
import jax.numpy as jnp
from jax import lax
from jax.experimental import pallas as pl
from jax.experimental.pallas import tpu as pltpu
from jax.experimental.pallas import tpu_sc as plsc

N = 100000
D = 128
B = 1024
S = 16

NC = 2
NS = 16
NW = NC * NS
L = 16

GPW = (B * S) // NW
CG = 8
NCH = GPW // CG
CROWS = CG * S
H1PW = (B * S) // NW
H2PW = B // NW
DCH = D // L


NBUF = 4


def _sc_body(features, samples0, samples1, nodes, nv0,
             agg0_o, h1_o, h2_o,
             idx_all, rows0, rows1, rows2, rows3, nv_v,
             agg0b, agg1b, agg2b, agg3b, h2idx, h2rows,
             sem0, sem1, sem2, sem3, ssem0, ssem1, ssem2, ssem3):
    wid = lax.axis_index("s") * NC + lax.axis_index("c")
    rows_bufs = (rows0, rows1, rows2, rows3)
    agg_bufs = (agg0b, agg1b, agg2b, agg3b)
    sems = (sem0, sem1, sem2, sem3)
    ssems = (ssem0, ssem1, ssem2, ssem3)

    b2 = wid * H2PW
    pltpu.sync_copy(nodes.at[pl.ds(b2, H2PW)], h2idx)
    pltpu.async_copy(features.at[h2idx], h2rows, sem0).wait()
    pltpu.sync_copy(h2rows, h2_o.at[pl.ds(b2, H2PW)])

    b1 = wid * H1PW
    pltpu.sync_copy(samples1.at[pl.ds(b1, 128 * 4)], idx_all.at[pl.ds(0, 128 * 4)])
    def h1_gather(c):
        pltpu.async_copy(
            features.at[idx_all.at[pl.ds(c * 128, 128)]],
            rows_bufs[c], sems[c])

    for c in range(4):
        h1_gather(c)
    for c in range(4):
        pltpu.make_async_copy(
            features.at[idx_all.at[pl.ds(c * 128, 128)]],
            rows_bufs[c], sems[c]).wait()
        pltpu.sync_copy(rows_bufs[c], h1_o.at[pl.ds(b1 + c * 128, 128)])

    gbase = wid * GPW
    pltpu.sync_copy(nv0.at[pl.ds(gbase, GPW)], nv_v)
    pltpu.sync_copy(samples0.at[pl.ds(gbase * S, GPW * S)], idx_all)

    def start_gather(c, p):
        pltpu.async_copy(
            features.at[idx_all.at[pl.ds(c * CROWS, CROWS)]],
            rows_bufs[p], sems[p])

    def wait_gather(c, p):
        pltpu.make_async_copy(
            features.at[idx_all.at[pl.ds(c * CROWS, CROWS)]],
            rows_bufs[p], sems[p]).wait()

    def start_store(c, p):
        pltpu.async_copy(
            agg_bufs[p], agg0_o.at[pl.ds(gbase + c * CG, CG)], ssems[p])

    def wait_store(c, p):
        pltpu.make_async_copy(
            agg_bufs[p], agg0_o.at[pl.ds(gbase + c * CG, CG)], ssems[p]).wait()

    def do_chunk(t, c, p, even):
        rows = rows_bufs[p]
        aggb = agg_bufs[p]
        wait_gather(c, p)

        @pl.when(t > 0)
        def _():
            wait_store(c - NBUF, p)

        nvvec = nv_v[pl.ds((c // 2) * 2 * CG, 2 * CG)]
        invvec = 1.0 / nvvec.astype(jnp.float32)
        for j in range(CG):
            lbase = j * S
            nv_j = jnp.where(even, nvvec[j], nvvec[j + CG])
            inv = jnp.where(even, invvec[j], invvec[j + CG])

            def acc_row(i, accs):
                r = lbase + i
                return tuple(accs[d] + rows[r, pl.ds(d * L, L)]
                             for d in range(DCH))

            accs = lax.fori_loop(
                0, nv_j, acc_row,
                tuple(jnp.zeros((L,), jnp.float32) for _ in range(DCH)))
            for d in range(DCH):
                aggb[j, pl.ds(d * L, L)] = accs[d] * inv
        start_store(c, p)

        @pl.when(c + NBUF < NCH)
        def _():
            start_gather(c + NBUF, p)

    for p in range(NBUF):
        start_gather(p, p)

    def agg_quad(t, carry):
        c = NBUF * t
        for q in range(NBUF):
            do_chunk(t, c + q, q, (q % 2) == 0)
        return carry
    lax.fori_loop(0, NCH // NBUF, agg_quad, 0)

    for p in range(NBUF):
        wait_store(NCH - NBUF + p, p)


@jax.jit
def _sc_gather(features, samples0, samples1, nodes, nv0):
    mesh = plsc.VectorSubcoreMesh(core_axis_name="c", subcore_axis_name="s",
                                  num_cores=NC, num_subcores=NS)
    return pl.kernel(
        _sc_body,
        out_type=(
            jax.ShapeDtypeStruct((B * S, D), jnp.float32),
            jax.ShapeDtypeStruct((B * S, D), jnp.float32),
            jax.ShapeDtypeStruct((B, D), jnp.float32),
        ),
        mesh=mesh,
        scratch_types=(
            [pltpu.VMEM((GPW * S,), jnp.int32)]
            + [pltpu.VMEM((CROWS, D), jnp.float32)] * NBUF
            + [pltpu.VMEM((GPW,), jnp.int32)]
            + [pltpu.VMEM((CG, D), jnp.float32)] * NBUF
            + [pltpu.VMEM((H2PW,), jnp.int32),
               pltpu.VMEM((H2PW, D), jnp.float32)]
            + [pltpu.SemaphoreType.DMA] * (2 * NBUF)
        ),
    )(features, samples0, samples1, nodes, nv0)


TCB = 8
RPB = (B * S) // TCB
GPB = B // TCB


def _tc_body(h1, agg0, h2, w1, w2, m1, inv1, out, agg1s, agg2s):
    t = pl.program_id(0)
    w1a = w1[:, :D]
    w1b = w1[:, D:]
    new1 = jnp.maximum(
        lax.dot_general(h1[:], w1a, (((1,), (1,)), ((), ())))
        + lax.dot_general(agg0[:], w1b, (((1,), (1,)), ((), ()))), 0.0)
    m = m1[:]
    iv = inv1[:]
    agg2s[pl.ds(t * GPB, GPB), :] = (
        jnp.sum((new1 * m).reshape(GPB, S, D), axis=1) * iv)
    agg1s[pl.ds(t * GPB, GPB), :] = (
        jnp.sum((h1[:] * m).reshape(GPB, S, D), axis=1) * iv)

    @pl.when(t == TCB - 1)
    def _():
        new2 = jnp.maximum(
            lax.dot_general(h2[:], w1a, (((1,), (1,)), ((), ())))
            + lax.dot_general(agg1s[:], w1b, (((1,), (1,)), ((), ()))), 0.0)
        out[:] = jnp.maximum(
            lax.dot_general(new2, w2[:, :D], (((1,), (1,)), ((), ())))
            + lax.dot_general(agg2s[:], w2[:, D:], (((1,), (1,)), ((), ()))),
            0.0)


@jax.jit
def _tc_combine(h1, agg0, h2, w1, w2, m1, inv1):
    return pl.pallas_call(
        _tc_body,
        grid=(TCB,),
        in_specs=[
            pl.BlockSpec((RPB, D), lambda t: (t, 0)),
            pl.BlockSpec((RPB, D), lambda t: (t, 0)),
            pl.BlockSpec((B, D), lambda t: (0, 0)),
            pl.BlockSpec((D, 2 * D), lambda t: (0, 0)),
            pl.BlockSpec((D, 2 * D), lambda t: (0, 0)),
            pl.BlockSpec((RPB, 1), lambda t: (t, 0)),
            pl.BlockSpec((GPB, 1), lambda t: (t, 0)),
        ],
        out_specs=pl.BlockSpec((B, D), lambda t: (0, 0)),
        out_shape=jax.ShapeDtypeStruct((B, D), jnp.float32),
        scratch_shapes=[
            pltpu.VMEM((B, D), jnp.float32),
            pltpu.VMEM((B, D), jnp.float32),
        ],
    )(h1, agg0, h2, w1, w2, m1, inv1)


def kernel(features, nodes, samples1, samples0, num_valid0, num_valid1, W1, W2):
    nodes = nodes.astype(jnp.int32)
    samples1 = samples1.astype(jnp.int32)
    samples0 = samples0.astype(jnp.int32)
    nv0 = num_valid0.reshape(-1).astype(jnp.int32)
    agg0, h1, h2 = _sc_gather(features, samples0, samples1, nodes, nv0)
    m1 = (jnp.arange(S)[None, :] < num_valid1).astype(jnp.float32)
    m1 = m1.reshape(B * S, 1)
    inv1 = 1.0 / num_valid1.astype(jnp.float32)
    return _tc_combine(h1, agg0, h2, W1, W2, m1, inv1)


# speedup vs baseline: 1.0188x; 1.0188x over previous
import functools

import jax
import jax.numpy as jnp
from jax import lax
from jax.experimental import pallas as pl
from jax.experimental.pallas import tpu as pltpu
from jax.experimental.pallas import tpu_sc as plsc

N = 100000
D = 128
B = 1024
S = 16

NC = 2
NS = 16
NW = NC * NS
L = 16

GPW = (B * S) // NW
CG = 8
NCH = GPW // CG
CROWS = CG * S
H1PW = (B * S) // NW
H2PW = B // NW
DCH = D // L


NBUF = 4


def _sc_body(features, samples0, samples1, nodes, nv0,
             agg0_o, h1_o, h2_o,
             idx_all, rows0, rows1, rows2, rows3, nv_v,
             agg0b, agg1b, agg2b, agg3b, h2idx, h2rows,
             sem0, sem1, sem2, sem3, ssem0, ssem1, ssem2, ssem3):
    wid = lax.axis_index("s") * NC + lax.axis_index("c")
    rows_bufs = (rows0, rows1, rows2, rows3)
    agg_bufs = (agg0b, agg1b, agg2b, agg3b)
    sems = (sem0, sem1, sem2, sem3)
    ssems = (ssem0, ssem1, ssem2, ssem3)

    b2 = wid * H2PW
    pltpu.sync_copy(nodes.at[pl.ds(b2, H2PW)], h2idx)
    pltpu.async_copy(features.at[h2idx], h2rows, sem0).wait()
    pltpu.sync_copy(h2rows, h2_o.at[pl.ds(b2, H2PW)])

    b1 = wid * H1PW
    pltpu.sync_copy(samples1.at[pl.ds(b1, 128 * 4)], idx_all.at[pl.ds(0, 128 * 4)])
    def h1_gather(c):
        pltpu.async_copy(
            features.at[idx_all.at[pl.ds(c * 128, 128)]],
            rows_bufs[c], sems[c])

    for c in range(4):
        h1_gather(c)
    for c in range(4):
        pltpu.make_async_copy(
            features.at[idx_all.at[pl.ds(c * 128, 128)]],
            rows_bufs[c], sems[c]).wait()
        pltpu.sync_copy(rows_bufs[c], h1_o.at[pl.ds(b1 + c * 128, 128)])

    gbase = wid * GPW
    pltpu.sync_copy(nv0.at[pl.ds(gbase, GPW)], nv_v)
    pltpu.sync_copy(samples0.at[pl.ds(gbase * S, GPW * S)], idx_all)

    def start_gather(c, p):
        pltpu.async_copy(
            features.at[idx_all.at[pl.ds(c * CROWS, CROWS)]],
            rows_bufs[p], sems[p])

    def wait_gather(c, p):
        pltpu.make_async_copy(
            features.at[idx_all.at[pl.ds(c * CROWS, CROWS)]],
            rows_bufs[p], sems[p]).wait()

    def start_store(c, p):
        pltpu.async_copy(
            agg_bufs[p], agg0_o.at[pl.ds(gbase + c * CG, CG)], ssems[p])

    def wait_store(c, p):
        pltpu.make_async_copy(
            agg_bufs[p], agg0_o.at[pl.ds(gbase + c * CG, CG)], ssems[p]).wait()

    def do_chunk(t, c, p, even):
        rows = rows_bufs[p]
        aggb = agg_bufs[p]
        wait_gather(c, p)

        @pl.when(t > 0)
        def _():
            wait_store(c - NBUF, p)

        nvvec = nv_v[pl.ds((c // 2) * 2 * CG, 2 * CG)]
        invvec = 1.0 / nvvec.astype(jnp.float32)
        for j in range(CG):
            lbase = j * S
            nv_j = jnp.where(even, nvvec[j], nvvec[j + CG])
            inv = jnp.where(even, invvec[j], invvec[j + CG])

            def acc_row(i, accs):
                r = lbase + i
                return tuple(accs[d] + rows[r, pl.ds(d * L, L)]
                             for d in range(DCH))

            accs = lax.fori_loop(
                0, nv_j, acc_row,
                tuple(jnp.zeros((L,), jnp.float32) for _ in range(DCH)))
            for d in range(DCH):
                aggb[j, pl.ds(d * L, L)] = accs[d] * inv
        start_store(c, p)

        @pl.when(c + NBUF < NCH)
        def _():
            start_gather(c + NBUF, p)

    for p in range(NBUF):
        start_gather(p, p)

    def agg_quad(t, carry):
        c = NBUF * t
        for q in range(NBUF):
            do_chunk(t, c + q, q, (q % 2) == 0)
        return carry
    lax.fori_loop(0, NCH // NBUF, agg_quad, 0)

    for p in range(NBUF):
        wait_store(NCH - NBUF + p, p)


@jax.jit
def _sc_gather(features, samples0, samples1, nodes, nv0):
    mesh = plsc.VectorSubcoreMesh(core_axis_name="c", subcore_axis_name="s",
                                  num_cores=NC, num_subcores=NS)
    return pl.kernel(
        _sc_body,
        out_type=(
            jax.ShapeDtypeStruct((B * S, D), jnp.float32),
            jax.ShapeDtypeStruct((B * S, D), jnp.float32),
            jax.ShapeDtypeStruct((B, D), jnp.float32),
        ),
        mesh=mesh,
        scratch_types=(
            [pltpu.VMEM((GPW * S,), jnp.int32)]
            + [pltpu.VMEM((CROWS, D), jnp.float32)] * NBUF
            + [pltpu.VMEM((GPW,), jnp.int32)]
            + [pltpu.VMEM((CG, D), jnp.float32)] * NBUF
            + [pltpu.VMEM((H2PW,), jnp.int32),
               pltpu.VMEM((H2PW, D), jnp.float32)]
            + [pltpu.SemaphoreType.DMA] * (2 * NBUF)
        ),
    )(features, samples0, samples1, nodes, nv0)


TCB = 4
RPB = (B * S) // TCB
GPB = B // TCB


def _tc_body(h1, agg0, h2, w1, w2, m1, inv1, out, agg1s, agg2s):
    t = pl.program_id(0)
    w1a = w1[:, :D]
    w1b = w1[:, D:]
    new1 = jnp.maximum(
        lax.dot_general(h1[:], w1a, (((1,), (1,)), ((), ())))
        + lax.dot_general(agg0[:], w1b, (((1,), (1,)), ((), ()))), 0.0)
    m = m1[:]
    iv = inv1[:]
    agg2s[pl.ds(t * GPB, GPB), :] = (
        jnp.sum((new1 * m).reshape(GPB, S, D), axis=1) * iv)
    agg1s[pl.ds(t * GPB, GPB), :] = (
        jnp.sum((h1[:] * m).reshape(GPB, S, D), axis=1) * iv)

    @pl.when(t == TCB - 1)
    def _():
        new2 = jnp.maximum(
            lax.dot_general(h2[:], w1a, (((1,), (1,)), ((), ())))
            + lax.dot_general(agg1s[:], w1b, (((1,), (1,)), ((), ()))), 0.0)
        out[:] = jnp.maximum(
            lax.dot_general(new2, w2[:, :D], (((1,), (1,)), ((), ())))
            + lax.dot_general(agg2s[:], w2[:, D:], (((1,), (1,)), ((), ()))),
            0.0)


@jax.jit
def _tc_combine(h1, agg0, h2, w1, w2, m1, inv1):
    return pl.pallas_call(
        _tc_body,
        grid=(TCB,),
        in_specs=[
            pl.BlockSpec((RPB, D), lambda t: (t, 0)),
            pl.BlockSpec((RPB, D), lambda t: (t, 0)),
            pl.BlockSpec((B, D), lambda t: (0, 0)),
            pl.BlockSpec((D, 2 * D), lambda t: (0, 0)),
            pl.BlockSpec((D, 2 * D), lambda t: (0, 0)),
            pl.BlockSpec((RPB, 1), lambda t: (t, 0)),
            pl.BlockSpec((GPB, 1), lambda t: (t, 0)),
        ],
        out_specs=pl.BlockSpec((B, D), lambda t: (0, 0)),
        out_shape=jax.ShapeDtypeStruct((B, D), jnp.float32),
        scratch_shapes=[
            pltpu.VMEM((B, D), jnp.float32),
            pltpu.VMEM((B, D), jnp.float32),
        ],
    )(h1, agg0, h2, w1, w2, m1, inv1)


def kernel(features, nodes, samples1, samples0, num_valid0, num_valid1, W1, W2):
    nodes = nodes.astype(jnp.int32)
    samples1 = samples1.astype(jnp.int32)
    samples0 = samples0.astype(jnp.int32)
    nv0 = num_valid0.reshape(-1).astype(jnp.int32)
    agg0, h1, h2 = _sc_gather(features, samples0, samples1, nodes, nv0)
    m1 = (jnp.arange(S)[None, :] < num_valid1).astype(jnp.float32)
    m1 = m1.reshape(B * S, 1)
    inv1 = 1.0 / num_valid1.astype(jnp.float32)
    return _tc_combine(h1, agg0, h2, W1, W2, m1, inv1)
